# baseline (speedup 1.0000x reference)
import jax
import jax.numpy as jnp
from jax import lax
from jax.experimental import pallas as pl
from jax.experimental.pallas import tpu as pltpu
from jax.experimental.pallas import tpu_sc as plsc

_NC = 2
_NS = 16
_K = 128


def _sc_agg_call(x2, src2, dst2, zf):
    _, _, C = x2.shape
    S = src2.shape[2]
    NP = zf.shape[0]
    R = NP // _NS
    mesh = plsc.VectorSubcoreMesh(core_axis_name="c", subcore_axis_name="s")

    SH = S // 2
    outs = [jax.ShapeDtypeStruct((_NC, NP, C), jnp.float32)]
    scratch = [
        pltpu.VMEM((SH, _K), jnp.int32),
        pltpu.VMEM((SH, _K), jnp.int32),
        pltpu.VMEM((_K, C), jnp.float32),
        pltpu.VMEM((_K, C), jnp.float32),
        pltpu.VMEM_SHARED((NP, C), jnp.float32),
        pltpu.SemaphoreType.DMA,
        pltpu.SemaphoreType.DMA,
        pltpu.SemaphoreType.DMA,
    ]

    def kbody(x2_r, src_r, dst_r, zf_r, agg_o, src_v, dst_v, gb0, gb1,
              acc_sh, sem0, sem1, ssem):
        c = lax.axis_index("c")
        s = lax.axis_index("s")
        row0 = s * R

        pltpu.sync_copy(zf_r.at[pl.ds(row0, R)], acc_sh.at[pl.ds(row0, R)])
        plsc.subcore_barrier()

        xsrc = x2_r.at[c]

        def step(jj, carry):
            j = jj * 2
            pltpu.sync_copy(gb0, acc_sh.at[dst_v.at[j]], add=True)
            pltpu.sync_copy(gb1, acc_sh.at[dst_v.at[j + 1]], add=True)
            return carry

        for phase in range(2):
            pltpu.sync_copy(src_r.at[c, s, pl.ds(phase * SH, SH)], src_v)
            pltpu.sync_copy(dst_r.at[c, s, pl.ds(phase * SH, SH)], dst_v)
            lax.fori_loop(0, SH // 2, step, 0)
        plsc.subcore_barrier()

        pltpu.sync_copy(acc_sh.at[pl.ds(row0, R)], agg_o.at[c, pl.ds(row0, R)])

    fn = pl.kernel(kbody, out_type=tuple(outs), mesh=mesh,
                   scratch_types=tuple(scratch))
    return fn(x2, src2, dst2, zf)[0]


def _sc_cnt_call(dstf, zc1):
    ept = dstf.shape[2]
    NR = zc1.shape[0]
    mesh = plsc.VectorSubcoreMesh(core_axis_name="c", subcore_axis_name="s")

    def kbody(dst_r, zc_r, cntp_o, dst_v, cnt_v):
        c = lax.axis_index("c")
        s = lax.axis_index("s")
        pltpu.sync_copy(dst_r.at[c, s], dst_v)
        pltpu.sync_copy(zc_r, cnt_v)
        onesv = jnp.ones((16,), jnp.float32)

        def step(i, carry):
            idx = dst_v[pl.ds(i * 16, 16)]
            plsc.addupdate_scatter(cnt_v, [idx >> 7, idx & 127], onesv)
            return carry

        lax.fori_loop(0, ept // 16, step, 0)
        pltpu.sync_copy(cnt_v, cntp_o.at[c, s])

    fn = pl.kernel(
        kbody,
        out_type=(jax.ShapeDtypeStruct((_NC, _NS, NR, 128), jnp.float32),),
        mesh=mesh,
        scratch_types=(
            pltpu.VMEM((ept,), jnp.int32),
            pltpu.VMEM((NR, 128), jnp.float32),
        ),
        compiler_params=pltpu.CompilerParams(needs_layout_passes=False),
    )
    return fn(dstf, zc1)[0]


def _tc_layer(agg, cnt, xdst2, wlt2, bl2, wrt2, lnw2, lnb2):
    _, N, C = xdst2.shape
    NP = agg.shape[1]

    NR = NP // 128

    def body(agg_r, cnt_r, x_r, wl_r, bl_r, wr_r, lw_r, lb_r, o_r):
        a3 = agg_r[0].reshape(NR, 128, C)
        cv = jnp.sum(cnt_r[0], axis=0)
        inv = 1.0 / jnp.maximum(cv, 1.0)
        m = (a3 * inv[:, :, None]).reshape(NP, C)[:N]
        h = (jnp.dot(m, wl_r[0], preferred_element_type=jnp.float32)
             + bl_r[0]
             + jnp.dot(x_r[0], wr_r[0], preferred_element_type=jnp.float32))
        mu = jnp.mean(h, axis=-1, keepdims=True)
        var = jnp.mean(jnp.square(h - mu), axis=-1, keepdims=True)
        y = (h - mu) * lax.rsqrt(var + 1e-5) * lw_r[0] + lb_r[0]
        o_r[0] = jnp.maximum(y, 0.0)

    return pl.pallas_call(
        body,
        grid=(2,),
        in_specs=[
            pl.BlockSpec((1, NP, C), lambda t: (t, 0, 0)),
            pl.BlockSpec((1, _NS, NP // 128, 128), lambda t: (t, 0, 0, 0)),
            pl.BlockSpec((1, N, C), lambda t: (t, 0, 0)),
            pl.BlockSpec((1, C, C), lambda t: (t, 0, 0)),
            pl.BlockSpec((1, 1, C), lambda t: (t, 0, 0)),
            pl.BlockSpec((1, C, C), lambda t: (t, 0, 0)),
            pl.BlockSpec((1, 1, C), lambda t: (t, 0, 0)),
            pl.BlockSpec((1, 1, C), lambda t: (t, 0, 0)),
        ],
        out_specs=pl.BlockSpec((1, N, C), lambda t: (t, 0, 0)),
        out_shape=jax.ShapeDtypeStruct((2, N, C), jnp.float32),
    )(agg, cnt, xdst2, wlt2, bl2, wrt2, lnw2, lnb2)


def kernel(x_user, x_item, ei_ui, ei_iu,
           Wl_0_ui, bl_0_ui, Wr_0_ui, Wl_0_iu, bl_0_iu, Wr_0_iu,
           ln_w_0_user, ln_b_0_user, ln_w_0_item, ln_b_0_item,
           Wl_1_ui, bl_1_ui, Wr_1_ui, Wl_1_iu, bl_1_iu, Wr_1_iu,
           ln_w_1_user, ln_b_1_user, ln_w_1_item, ln_b_1_item):
    N, C = x_user.shape
    E = ei_ui.shape[1]
    ept = -(-E // (_NS * 4 * _K)) * 4 * _K
    S = ept // _K
    tot = ept * _NS
    NP = ((N + 1 + 127) // 128) * 128

    def prep(ei):
        pad = tot - E
        src = jnp.concatenate([ei[0], jnp.zeros((pad,), jnp.int32)])
        dst = jnp.concatenate([ei[1], jnp.full((pad,), N, jnp.int32)])
        return src.reshape(_NS, S, _K), dst.reshape(_NS, S, _K)

    order_ui = jnp.argsort(ei_ui[0])
    order_iu = jnp.argsort(ei_iu[0])
    ei_ui = ei_ui[:, order_ui]
    ei_iu = ei_iu[:, order_iu]
    su, du = prep(ei_ui)
    si, di = prep(ei_iu)
    src2 = jnp.stack([su, si])
    dst2 = jnp.stack([du, di])
    zf = jnp.zeros((NP, C), jnp.float32)
    zc1 = jnp.zeros((NP // 128, 128), jnp.float32)

    cnt = _sc_cnt_call(dst2.reshape(_NC, _NS, -1), zc1)
    agg0 = _sc_agg_call(jnp.stack([x_user, x_item]), src2, dst2, zf)
    out0 = _tc_layer(
        agg0, cnt, jnp.stack([x_item, x_user]),
        jnp.stack([Wl_0_ui.T, Wl_0_iu.T]),
        jnp.stack([bl_0_ui, bl_0_iu])[:, None, :],
        jnp.stack([Wr_0_ui.T, Wr_0_iu.T]),
        jnp.stack([ln_w_0_item, ln_w_0_user])[:, None, :],
        jnp.stack([ln_b_0_item, ln_b_0_user])[:, None, :],
    )
    x_item1, x_user1 = out0[0], out0[1]

    agg1 = _sc_agg_call(jnp.stack([x_user1, x_item1]), src2, dst2, zf)
    out1 = _tc_layer(
        agg1, cnt, jnp.stack([x_item1, x_user1]),
        jnp.stack([Wl_1_ui.T, Wl_1_iu.T]),
        jnp.stack([bl_1_ui, bl_1_iu])[:, None, :],
        jnp.stack([Wr_1_ui.T, Wr_1_iu.T]),
        jnp.stack([ln_w_1_item, ln_w_1_user])[:, None, :],
        jnp.stack([ln_b_1_item, ln_b_1_user])[:, None, :],
    )
    return (out1[1], out1[0])


# speedup vs baseline: 1.0917x; 1.0917x over previous
import jax
import jax.numpy as jnp
from jax import lax
from jax.experimental import pallas as pl
from jax.experimental.pallas import tpu as pltpu
from jax.experimental.pallas import tpu_sc as plsc

_NC = 2
_NS = 16
_K = 128


def _sc_agg_call(x2, src2, dst2, zf):
    _, _, C = x2.shape
    S = dst2.shape[2]
    NP = zf.shape[0]
    R = NP // _NS
    mesh = plsc.VectorSubcoreMesh(core_axis_name="c", subcore_axis_name="s")

    SH = S // 2
    outs = [jax.ShapeDtypeStruct((_NC, NP, C), jnp.float32)]
    scratch = [
        pltpu.VMEM((SH + 8, _K), jnp.int32),
        pltpu.VMEM((SH, _K), jnp.int32),
        pltpu.VMEM((_K, C), jnp.float32),
        pltpu.VMEM((_K, C), jnp.float32),
        pltpu.VMEM_SHARED((NP, C), jnp.float32),
        pltpu.SemaphoreType.DMA,
        pltpu.SemaphoreType.DMA,
    ]

    def kbody(x2_r, src_r, dst_r, zf_r, agg_o, src_v, dst_v, gb0, gb1,
              acc_sh, sem0, sem1):
        c = lax.axis_index("c")
        s = lax.axis_index("s")
        row0 = s * R

        pltpu.sync_copy(zf_r.at[pl.ds(row0, R)], acc_sh.at[pl.ds(row0, R)])
        plsc.subcore_barrier()

        xsrc = x2_r.at[c]

        def step(jj, carry):
            j = jj * 2
            pltpu.make_async_copy(xsrc.at[src_v.at[0]], gb0, sem0).wait()
            pltpu.async_copy(xsrc.at[src_v.at[j + 1]], gb1, sem1)
            pltpu.sync_copy(gb0, acc_sh.at[dst_v.at[j]], add=True)
            pltpu.make_async_copy(xsrc.at[src_v.at[0]], gb1, sem1).wait()
            pltpu.async_copy(xsrc.at[src_v.at[j + 2]], gb0, sem0)
            pltpu.sync_copy(gb1, acc_sh.at[dst_v.at[j + 1]], add=True)
            return carry

        for phase in range(2):
            pltpu.sync_copy(src_r.at[c, s, pl.ds(phase * SH, SH + 8)], src_v)
            pltpu.sync_copy(dst_r.at[c, s, pl.ds(phase * SH, SH)], dst_v)
            pltpu.async_copy(xsrc.at[src_v.at[0]], gb0, sem0)
            lax.fori_loop(0, SH // 2, step, 0)
            pltpu.make_async_copy(xsrc.at[src_v.at[0]], gb0, sem0).wait()
        plsc.subcore_barrier()

        pltpu.sync_copy(acc_sh.at[pl.ds(row0, R)], agg_o.at[c, pl.ds(row0, R)])

    fn = pl.kernel(kbody, out_type=tuple(outs), mesh=mesh,
                   scratch_types=tuple(scratch))
    return fn(x2, src2, dst2, zf)[0]


def _sc_cnt_call(dstf, zc1):
    ept = dstf.shape[2]
    NR = zc1.shape[0]
    mesh = plsc.VectorSubcoreMesh(core_axis_name="c", subcore_axis_name="s")

    def kbody(dst_r, zc_r, cntp_o, dst_v, cnt_v):
        c = lax.axis_index("c")
        s = lax.axis_index("s")
        pltpu.sync_copy(dst_r.at[c, s], dst_v)
        pltpu.sync_copy(zc_r, cnt_v)
        onesv = jnp.ones((16,), jnp.float32)

        def step(i, carry):
            idx = dst_v[pl.ds(i * 16, 16)]
            plsc.addupdate_scatter(cnt_v, [idx >> 7, idx & 127], onesv)
            return carry

        lax.fori_loop(0, ept // 16, step, 0)
        pltpu.sync_copy(cnt_v, cntp_o.at[c, s])

    fn = pl.kernel(
        kbody,
        out_type=(jax.ShapeDtypeStruct((_NC, _NS, NR, 128), jnp.float32),),
        mesh=mesh,
        scratch_types=(
            pltpu.VMEM((ept,), jnp.int32),
            pltpu.VMEM((NR, 128), jnp.float32),
        ),
        compiler_params=pltpu.CompilerParams(needs_layout_passes=False),
    )
    return fn(dstf, zc1)[0]


def _tc_layer(agg, cnt, xdst2, wlt2, bl2, wrt2, lnw2, lnb2):
    _, N, C = xdst2.shape
    NP = agg.shape[1]

    NR = NP // 128

    def body(agg_r, cnt_r, x_r, wl_r, bl_r, wr_r, lw_r, lb_r, o_r):
        a3 = agg_r[0].reshape(NR, 128, C)
        cv = jnp.sum(cnt_r[0], axis=0)
        inv = 1.0 / jnp.maximum(cv, 1.0)
        m = (a3 * inv[:, :, None]).reshape(NP, C)[:N]
        h = (jnp.dot(m, wl_r[0], preferred_element_type=jnp.float32)
             + bl_r[0]
             + jnp.dot(x_r[0], wr_r[0], preferred_element_type=jnp.float32))
        mu = jnp.mean(h, axis=-1, keepdims=True)
        var = jnp.mean(jnp.square(h - mu), axis=-1, keepdims=True)
        y = (h - mu) * lax.rsqrt(var + 1e-5) * lw_r[0] + lb_r[0]
        o_r[0] = jnp.maximum(y, 0.0)

    return pl.pallas_call(
        body,
        grid=(2,),
        in_specs=[
            pl.BlockSpec((1, NP, C), lambda t: (t, 0, 0)),
            pl.BlockSpec((1, _NS, NP // 128, 128), lambda t: (t, 0, 0, 0)),
            pl.BlockSpec((1, N, C), lambda t: (t, 0, 0)),
            pl.BlockSpec((1, C, C), lambda t: (t, 0, 0)),
            pl.BlockSpec((1, 1, C), lambda t: (t, 0, 0)),
            pl.BlockSpec((1, C, C), lambda t: (t, 0, 0)),
            pl.BlockSpec((1, 1, C), lambda t: (t, 0, 0)),
            pl.BlockSpec((1, 1, C), lambda t: (t, 0, 0)),
        ],
        out_specs=pl.BlockSpec((1, N, C), lambda t: (t, 0, 0)),
        out_shape=jax.ShapeDtypeStruct((2, N, C), jnp.float32),
    )(agg, cnt, xdst2, wlt2, bl2, wrt2, lnw2, lnb2)


def kernel(x_user, x_item, ei_ui, ei_iu,
           Wl_0_ui, bl_0_ui, Wr_0_ui, Wl_0_iu, bl_0_iu, Wr_0_iu,
           ln_w_0_user, ln_b_0_user, ln_w_0_item, ln_b_0_item,
           Wl_1_ui, bl_1_ui, Wr_1_ui, Wl_1_iu, bl_1_iu, Wr_1_iu,
           ln_w_1_user, ln_b_1_user, ln_w_1_item, ln_b_1_item):
    N, C = x_user.shape
    E = ei_ui.shape[1]
    ept = -(-E // (_NS * 4 * _K)) * 4 * _K
    S = ept // _K
    tot = ept * _NS
    NP = ((N + 1 + 127) // 128) * 128

    def prep(ei):
        pad = tot - E
        src = jnp.concatenate([ei[0], jnp.zeros((pad,), jnp.int32)])
        dst = jnp.concatenate([ei[1], jnp.full((pad,), N, jnp.int32)])
        src = src.reshape(_NS, S, _K)
        src = jnp.concatenate([src, jnp.zeros((_NS, 8, _K), jnp.int32)], 1)
        return src, dst.reshape(_NS, S, _K)

    su, du = prep(ei_ui)
    si, di = prep(ei_iu)
    src2 = jnp.stack([su, si])
    dst2 = jnp.stack([du, di])
    zf = jnp.zeros((NP, C), jnp.float32)
    zc1 = jnp.zeros((NP // 128, 128), jnp.float32)

    cnt = _sc_cnt_call(dst2.reshape(_NC, _NS, -1), zc1)
    agg0 = _sc_agg_call(jnp.stack([x_user, x_item]), src2, dst2, zf)
    out0 = _tc_layer(
        agg0, cnt, jnp.stack([x_item, x_user]),
        jnp.stack([Wl_0_ui.T, Wl_0_iu.T]),
        jnp.stack([bl_0_ui, bl_0_iu])[:, None, :],
        jnp.stack([Wr_0_ui.T, Wr_0_iu.T]),
        jnp.stack([ln_w_0_item, ln_w_0_user])[:, None, :],
        jnp.stack([ln_b_0_item, ln_b_0_user])[:, None, :],
    )
    x_item1, x_user1 = out0[0], out0[1]

    agg1 = _sc_agg_call(jnp.stack([x_user1, x_item1]), src2, dst2, zf)
    out1 = _tc_layer(
        agg1, cnt, jnp.stack([x_item1, x_user1]),
        jnp.stack([Wl_1_ui.T, Wl_1_iu.T]),
        jnp.stack([bl_1_ui, bl_1_iu])[:, None, :],
        jnp.stack([Wr_1_ui.T, Wr_1_iu.T]),
        jnp.stack([ln_w_1_item, ln_w_1_user])[:, None, :],
        jnp.stack([ln_b_1_item, ln_b_1_user])[:, None, :],
    )
    return (out1[1], out1[0])


# speedup vs baseline: 1.1505x; 1.0539x over previous
import jax
import jax.numpy as jnp
from jax import lax
from jax.experimental import pallas as pl
from jax.experimental.pallas import tpu as pltpu
from jax.experimental.pallas import tpu_sc as plsc

_NC = 2
_NS = 16
_K = 128


def _sc_agg_call(x2, src2, dst2, zf):
    _, _, C = x2.shape
    S = src2.shape[2]
    NP = zf.shape[0]
    R = NP // _NS
    mesh = plsc.VectorSubcoreMesh(core_axis_name="c", subcore_axis_name="s")

    outs = [jax.ShapeDtypeStruct((_NC, NP, C), jnp.float32)]
    scratch = [
        pltpu.VMEM((S, _K), jnp.int32),
        pltpu.VMEM((S, _K), jnp.int32),
        pltpu.VMEM((_K, C), jnp.float32),
        pltpu.VMEM_SHARED((NP, C), jnp.float32),
        pltpu.SemaphoreType.DMA,
    ]

    def kbody(x2_r, src_r, dst_r, zf_r, agg_o, src_v, dst_v, gbuf, acc_sh,
              sem):
        c = lax.axis_index("c")
        s = lax.axis_index("s")
        row0 = s * R

        pltpu.sync_copy(src_r.at[c, s], src_v)
        pltpu.sync_copy(dst_r.at[c, s], dst_v)
        pltpu.sync_copy(zf_r.at[pl.ds(row0, R)], acc_sh.at[pl.ds(row0, R)])
        plsc.subcore_barrier()

        xsrc = x2_r.at[c]

        def step(j, carry):
            pltpu.async_copy(xsrc.at[src_v.at[j]], gbuf, sem).wait()
            pltpu.sync_copy(gbuf, acc_sh.at[dst_v.at[j]], add=True)
            return carry

        lax.fori_loop(0, S, step, 0)
        plsc.subcore_barrier()

        pltpu.sync_copy(acc_sh.at[pl.ds(row0, R)], agg_o.at[c, pl.ds(row0, R)])

    fn = pl.kernel(kbody, out_type=tuple(outs), mesh=mesh,
                   scratch_types=tuple(scratch))
    return fn(x2, src2, dst2, zf)[0]


def _sc_cnt_call(dstf, zc1):
    ept = dstf.shape[2]
    NR = zc1.shape[0]
    mesh = plsc.VectorSubcoreMesh(core_axis_name="c", subcore_axis_name="s")

    def kbody(dst_r, zc_r, cntp_o, dst_v, cnt_v):
        c = lax.axis_index("c")
        s = lax.axis_index("s")
        pltpu.sync_copy(dst_r.at[c, s], dst_v)
        pltpu.sync_copy(zc_r, cnt_v)
        onesv = jnp.ones((16,), jnp.float32)

        def step(i, carry):
            idx = dst_v[pl.ds(i * 16, 16)]
            plsc.addupdate_scatter(cnt_v, [idx >> 7, idx & 127], onesv)
            return carry

        lax.fori_loop(0, ept // 16, step, 0)
        pltpu.sync_copy(cnt_v, cntp_o.at[c, s])

    fn = pl.kernel(
        kbody,
        out_type=(jax.ShapeDtypeStruct((_NC, _NS, NR, 128), jnp.float32),),
        mesh=mesh,
        scratch_types=(
            pltpu.VMEM((ept,), jnp.int32),
            pltpu.VMEM((NR, 128), jnp.float32),
        ),
        compiler_params=pltpu.CompilerParams(needs_layout_passes=False),
    )
    return fn(dstf, zc1)[0]


def _tc_r(xdst2, wrt2, bl2):
    _, N, C = xdst2.shape

    def body(x_r, wr_r, bl_r, o_r):
        o_r[0] = (jnp.dot(x_r[0], wr_r[0], preferred_element_type=jnp.float32)
                  + bl_r[0])

    return pl.pallas_call(
        body,
        grid=(2,),
        in_specs=[
            pl.BlockSpec((1, N, C), lambda t: (t, 0, 0)),
            pl.BlockSpec((1, C, C), lambda t: (t, 0, 0)),
            pl.BlockSpec((1, 1, C), lambda t: (t, 0, 0)),
        ],
        out_specs=pl.BlockSpec((1, N, C), lambda t: (t, 0, 0)),
        out_shape=jax.ShapeDtypeStruct((2, N, C), jnp.float32),
    )(xdst2, wrt2, bl2)


def _tc_c(agg, cnt, r2, wlt2, lnw2, lnb2):
    _, N, C = r2.shape
    NP = agg.shape[1]
    NR = NP // 128

    def body(agg_r, cnt_r, r_r, wl_r, lw_r, lb_r, o_r):
        a3 = agg_r[0].reshape(NR, 128, C)
        cv = jnp.sum(cnt_r[0], axis=0)
        inv = 1.0 / jnp.maximum(cv, 1.0)
        m = (a3 * inv[:, :, None]).reshape(NP, C)[:N]
        h = (jnp.dot(m, wl_r[0], preferred_element_type=jnp.float32)
             + r_r[0])
        mu = jnp.mean(h, axis=-1, keepdims=True)
        var = jnp.mean(jnp.square(h - mu), axis=-1, keepdims=True)
        y = (h - mu) * lax.rsqrt(var + 1e-5) * lw_r[0] + lb_r[0]
        o_r[0] = jnp.maximum(y, 0.0)

    return pl.pallas_call(
        body,
        grid=(2,),
        in_specs=[
            pl.BlockSpec((1, NP, C), lambda t: (t, 0, 0)),
            pl.BlockSpec((1, _NS, NP // 128, 128), lambda t: (t, 0, 0, 0)),
            pl.BlockSpec((1, N, C), lambda t: (t, 0, 0)),
            pl.BlockSpec((1, C, C), lambda t: (t, 0, 0)),
            pl.BlockSpec((1, 1, C), lambda t: (t, 0, 0)),
            pl.BlockSpec((1, 1, C), lambda t: (t, 0, 0)),
        ],
        out_specs=pl.BlockSpec((1, N, C), lambda t: (t, 0, 0)),
        out_shape=jax.ShapeDtypeStruct((2, N, C), jnp.float32),
    )(agg, cnt, r2, wlt2, lnw2, lnb2)


def kernel(x_user, x_item, ei_ui, ei_iu,
           Wl_0_ui, bl_0_ui, Wr_0_ui, Wl_0_iu, bl_0_iu, Wr_0_iu,
           ln_w_0_user, ln_b_0_user, ln_w_0_item, ln_b_0_item,
           Wl_1_ui, bl_1_ui, Wr_1_ui, Wl_1_iu, bl_1_iu, Wr_1_iu,
           ln_w_1_user, ln_b_1_user, ln_w_1_item, ln_b_1_item):
    N, C = x_user.shape
    E = ei_ui.shape[1]
    ept = -(-E // (_NS * 4 * _K)) * 4 * _K
    S = ept // _K
    tot = ept * _NS
    NP = ((N + 1 + 127) // 128) * 128

    def prep(ei):
        pad = tot - E
        src = jnp.concatenate([ei[0], jnp.zeros((pad,), jnp.int32)])
        dst = jnp.concatenate([ei[1], jnp.full((pad,), N, jnp.int32)])
        return src.reshape(_NS, S, _K), dst.reshape(_NS, S, _K)

    su, du = prep(ei_ui)
    si, di = prep(ei_iu)
    src2 = jnp.stack([su, si])
    dst2 = jnp.stack([du, di])
    zf = jnp.zeros((NP, C), jnp.float32)
    zc1 = jnp.zeros((NP // 128, 128), jnp.float32)

    cnt = _sc_cnt_call(dst2.reshape(_NC, _NS, -1), zc1)
    agg0 = _sc_agg_call(jnp.stack([x_user, x_item]), src2, dst2, zf)
    r0 = _tc_r(jnp.stack([x_item, x_user]),
               jnp.stack([Wr_0_ui.T, Wr_0_iu.T]),
               jnp.stack([bl_0_ui, bl_0_iu])[:, None, :])
    out0 = _tc_c(
        agg0, cnt, r0,
        jnp.stack([Wl_0_ui.T, Wl_0_iu.T]),
        jnp.stack([ln_w_0_item, ln_w_0_user])[:, None, :],
        jnp.stack([ln_b_0_item, ln_b_0_user])[:, None, :],
    )
    x_item1, x_user1 = out0[0], out0[1]

    agg1 = _sc_agg_call(jnp.stack([x_user1, x_item1]), src2, dst2, zf)
    r1 = _tc_r(jnp.stack([x_item1, x_user1]),
               jnp.stack([Wr_1_ui.T, Wr_1_iu.T]),
               jnp.stack([bl_1_ui, bl_1_iu])[:, None, :])
    out1 = _tc_c(
        agg1, cnt, r1,
        jnp.stack([Wl_1_ui.T, Wl_1_iu.T]),
        jnp.stack([ln_w_1_item, ln_w_1_user])[:, None, :],
        jnp.stack([ln_b_1_item, ln_b_1_user])[:, None, :],
    )
    return (out1[1], out1[0])


# speedup vs baseline: 1.1701x; 1.0171x over previous
import jax
import jax.numpy as jnp
from jax import lax
from jax.experimental import pallas as pl
from jax.experimental.pallas import tpu as pltpu
from jax.experimental.pallas import tpu_sc as plsc

_NC = 2
_NS = 16
_K = 128


def _sc_agg_call(x2, src2, dst2, zf):
    _, _, C = x2.shape
    S = src2.shape[2]
    NP = zf.shape[0]
    R = NP // _NS
    mesh = plsc.VectorSubcoreMesh(core_axis_name="c", subcore_axis_name="s")

    outs = [jax.ShapeDtypeStruct((_NC, NP, C), jnp.float32)]
    scratch = [
        pltpu.VMEM((S, _K), jnp.int32),
        pltpu.VMEM((S, _K), jnp.int32),
        pltpu.VMEM((_K, C), jnp.float32),
        pltpu.VMEM_SHARED((NP, C), jnp.float32),
        pltpu.SemaphoreType.DMA,
    ]

    def kbody(x2_r, src_r, dst_r, zf_r, agg_o, src_v, dst_v, gbuf, acc_sh,
              sem):
        c = lax.axis_index("c")
        s = lax.axis_index("s")
        row0 = s * R

        pltpu.sync_copy(src_r.at[c, s], src_v)
        pltpu.sync_copy(dst_r.at[c, s], dst_v)
        pltpu.sync_copy(zf_r.at[pl.ds(row0, R)], acc_sh.at[pl.ds(row0, R)])
        plsc.subcore_barrier()

        xsrc = x2_r.at[c]

        def step(j, carry):
            pltpu.async_copy(xsrc.at[src_v.at[j]], gbuf, sem).wait()
            pltpu.sync_copy(gbuf, acc_sh.at[dst_v.at[j]], add=True)
            return carry

        lax.fori_loop(0, S, step, 0)
        plsc.subcore_barrier()

        pltpu.sync_copy(acc_sh.at[pl.ds(row0, R)], agg_o.at[c, pl.ds(row0, R)])

    fn = pl.kernel(kbody, out_type=tuple(outs), mesh=mesh,
                   scratch_types=tuple(scratch))
    return fn(x2, src2, dst2, zf)[0]


def _sc_cnt_call(dstf, zc1):
    ept = dstf.shape[2]
    NR = zc1.shape[0]
    mesh = plsc.VectorSubcoreMesh(core_axis_name="c", subcore_axis_name="s")

    def kbody(dst_r, zc_r, cntp_o, dst_v, cnt_v):
        c = lax.axis_index("c")
        s = lax.axis_index("s")
        pltpu.sync_copy(dst_r.at[c, s], dst_v)
        pltpu.sync_copy(zc_r, cnt_v)
        onesv = jnp.ones((16,), jnp.float32)

        def step(i, carry):
            idx = dst_v[pl.ds(i * 16, 16)]
            plsc.addupdate_scatter(cnt_v, [idx >> 7, idx & 127], onesv)
            return carry

        lax.fori_loop(0, ept // 16, step, 0)
        pltpu.sync_copy(cnt_v, cntp_o.at[c, s])

    fn = pl.kernel(
        kbody,
        out_type=(jax.ShapeDtypeStruct((_NC, _NS, NR, 128), jnp.float32),),
        mesh=mesh,
        scratch_types=(
            pltpu.VMEM((ept,), jnp.int32),
            pltpu.VMEM((NR, 128), jnp.float32),
        ),
        compiler_params=pltpu.CompilerParams(needs_layout_passes=False),
    )
    return fn(dstf, zc1)[0]


def _tc_layer(agg, cnt, xdst2, wlt2, bl2, wrt2, lnw2, lnb2):
    _, N, C = xdst2.shape
    NP = agg.shape[1]

    NR = NP // 128

    def body(agg_r, cnt_r, x_r, wl_r, bl_r, wr_r, lw_r, lb_r, o_r):
        a3 = agg_r[0].reshape(NR, 128, C)
        cv = jnp.sum(cnt_r[0], axis=0)
        inv = 1.0 / jnp.maximum(cv, 1.0)
        m = (a3 * inv[:, :, None]).reshape(NP, C)[:N]
        h = (jnp.dot(m, wl_r[0], preferred_element_type=jnp.float32)
             + bl_r[0]
             + jnp.dot(x_r[0], wr_r[0], preferred_element_type=jnp.float32))
        mu = jnp.mean(h, axis=-1, keepdims=True)
        var = jnp.mean(jnp.square(h - mu), axis=-1, keepdims=True)
        y = (h - mu) * lax.rsqrt(var + 1e-5) * lw_r[0] + lb_r[0]
        o_r[0] = jnp.maximum(y, 0.0)

    return pl.pallas_call(
        body,
        grid=(2,),
        in_specs=[
            pl.BlockSpec((1, NP, C), lambda t: (t, 0, 0)),
            pl.BlockSpec((1, _NS, NP // 128, 128), lambda t: (t, 0, 0, 0)),
            pl.BlockSpec((1, N, C), lambda t: (t, 0, 0)),
            pl.BlockSpec((1, C, C), lambda t: (t, 0, 0)),
            pl.BlockSpec((1, 1, C), lambda t: (t, 0, 0)),
            pl.BlockSpec((1, C, C), lambda t: (t, 0, 0)),
            pl.BlockSpec((1, 1, C), lambda t: (t, 0, 0)),
            pl.BlockSpec((1, 1, C), lambda t: (t, 0, 0)),
        ],
        out_specs=pl.BlockSpec((1, N, C), lambda t: (t, 0, 0)),
        out_shape=jax.ShapeDtypeStruct((2, N, C), jnp.float32),
    )(agg, cnt, xdst2, wlt2, bl2, wrt2, lnw2, lnb2)


def kernel(x_user, x_item, ei_ui, ei_iu,
           Wl_0_ui, bl_0_ui, Wr_0_ui, Wl_0_iu, bl_0_iu, Wr_0_iu,
           ln_w_0_user, ln_b_0_user, ln_w_0_item, ln_b_0_item,
           Wl_1_ui, bl_1_ui, Wr_1_ui, Wl_1_iu, bl_1_iu, Wr_1_iu,
           ln_w_1_user, ln_b_1_user, ln_w_1_item, ln_b_1_item):
    N, C = x_user.shape
    E = ei_ui.shape[1]
    ept = -(-E // (_NS * 4 * _K)) * 4 * _K
    S = ept // _K
    tot = ept * _NS
    NP = ((N + 1 + 127) // 128) * 128

    def prep(ei):
        pad = tot - E
        src = jnp.concatenate([ei[0], jnp.zeros((pad,), jnp.int32)])
        dst = jnp.concatenate([ei[1], jnp.full((pad,), N, jnp.int32)])
        return src.reshape(_NS, S, _K), dst.reshape(_NS, S, _K)

    su, du = prep(ei_ui)
    si, di = prep(ei_iu)
    src2 = jnp.stack([su, si])
    dst2 = jnp.stack([du, di])
    zf = jnp.zeros((NP, C), jnp.float32)
    zc1 = jnp.zeros((NP // 128, 128), jnp.float32)

    cnt = _sc_cnt_call(dst2.reshape(_NC, _NS, -1), zc1)
    agg0 = _sc_agg_call(jnp.stack([x_user, x_item]), src2, dst2, zf)
    out0 = _tc_layer(
        agg0, cnt, jnp.stack([x_item, x_user]),
        jnp.stack([Wl_0_ui.T, Wl_0_iu.T]),
        jnp.stack([bl_0_ui, bl_0_iu])[:, None, :],
        jnp.stack([Wr_0_ui.T, Wr_0_iu.T]),
        jnp.stack([ln_w_0_item, ln_w_0_user])[:, None, :],
        jnp.stack([ln_b_0_item, ln_b_0_user])[:, None, :],
    )
    x_item1, x_user1 = out0[0], out0[1]

    agg1 = _sc_agg_call(jnp.stack([x_user1, x_item1]), src2, dst2, zf)
    out1 = _tc_layer(
        agg1, cnt, jnp.stack([x_item1, x_user1]),
        jnp.stack([Wl_1_ui.T, Wl_1_iu.T]),
        jnp.stack([bl_1_ui, bl_1_iu])[:, None, :],
        jnp.stack([Wr_1_ui.T, Wr_1_iu.T]),
        jnp.stack([ln_w_1_item, ln_w_1_user])[:, None, :],
        jnp.stack([ln_b_1_item, ln_b_1_user])[:, None, :],
    )
    return (out1[1], out1[0])


# speedup vs baseline: 1.4976x; 1.2798x over previous
import jax
import jax.numpy as jnp
from jax import lax
from jax.experimental import pallas as pl
from jax.experimental.pallas import tpu as pltpu
from jax.experimental.pallas import tpu_sc as plsc

_NC = 2
_NS = 16
_K = 128


def _sc_agg_call(x2, src2, dst2, zf):
    _, _, C = x2.shape
    S = src2.shape[2]
    NP = zf.shape[0]
    R = NP // _NS
    mesh = plsc.VectorSubcoreMesh(core_axis_name="c", subcore_axis_name="s")

    outs = [jax.ShapeDtypeStruct((_NC, NP, C), jnp.float32)]
    scratch = [
        pltpu.VMEM((S, _K), jnp.int32),
        pltpu.VMEM((S, _K), jnp.int32),
        pltpu.VMEM((_K, C), jnp.float32),
        pltpu.VMEM_SHARED((NP, C), jnp.float32),
        pltpu.SemaphoreType.DMA,
    ]

    def kbody(x2_r, src_r, dst_r, zf_r, agg_o, src_v, dst_v, gbuf, acc_sh,
              sem):
        c = lax.axis_index("c")
        s = lax.axis_index("s")
        row0 = s * R

        pltpu.sync_copy(src_r.at[c, s], src_v)
        pltpu.sync_copy(dst_r.at[c, s], dst_v)
        pltpu.sync_copy(zf_r.at[pl.ds(row0, R)], acc_sh.at[pl.ds(row0, R)])
        plsc.subcore_barrier()

        xsrc = x2_r.at[c]

        def step(j, carry):
            pltpu.async_copy(xsrc.at[src_v.at[j]], gbuf, sem).wait()
            pltpu.sync_copy(gbuf, acc_sh.at[dst_v.at[j]], add=True)
            return carry

        lax.fori_loop(0, S, step, 0)
        plsc.subcore_barrier()

        pltpu.sync_copy(acc_sh.at[pl.ds(row0, R)], agg_o.at[c, pl.ds(row0, R)])

    fn = pl.kernel(kbody, out_type=tuple(outs), mesh=mesh,
                   scratch_types=tuple(scratch))
    return fn(x2, src2, dst2, zf)[0]


def _sc_cnt_call(dstf, zc1):
    ept = dstf.shape[2]
    NR = zc1.shape[0]
    mesh = plsc.VectorSubcoreMesh(core_axis_name="c", subcore_axis_name="s")

    def kbody(dst_r, zc_r, cntp_o, dst_v, cnt_v):
        c = lax.axis_index("c")
        s = lax.axis_index("s")
        pltpu.sync_copy(dst_r.at[c, s], dst_v)
        pltpu.sync_copy(zc_r, cnt_v)
        onesv = jnp.ones((16,), jnp.float32)

        def step(i, carry):
            idx = dst_v[pl.ds(i * 16, 16)]
            plsc.addupdate_scatter(cnt_v, [idx >> 7, idx & 127], onesv)
            return carry

        lax.fori_loop(0, ept // 16, step, 0)
        pltpu.sync_copy(cnt_v, cntp_o.at[c, s])

    fn = pl.kernel(
        kbody,
        out_type=(jax.ShapeDtypeStruct((_NC, _NS, NR, 128), jnp.float32),),
        mesh=mesh,
        scratch_types=(
            pltpu.VMEM((ept,), jnp.int32),
            pltpu.VMEM((NR, 128), jnp.float32),
        ),
        compiler_params=pltpu.CompilerParams(needs_layout_passes=False),
    )
    return fn(dstf, zc1)[0]


def _tc_layer(agg, cnt, xdst2, wlt2, bl2, wrt2, lnw2, lnb2):
    _, N, C = xdst2.shape
    NP = agg.shape[1]

    NR = NP // 128

    def body(agg_r, cnt_r, x_r, wl_r, bl_r, wr_r, lw_r, lb_r, o_r):
        a3 = agg_r[0].reshape(NR, 128, C)
        cv = jnp.sum(cnt_r[0], axis=0)
        inv = 1.0 / jnp.maximum(cv, 1.0)
        m = (a3 * inv[:, :, None]).reshape(NP, C)[:N]
        h = (jnp.dot(m, wl_r[0], preferred_element_type=jnp.float32)
             + bl_r[0]
             + jnp.dot(x_r[0], wr_r[0], preferred_element_type=jnp.float32))
        mu = jnp.mean(h, axis=-1, keepdims=True)
        var = jnp.mean(jnp.square(h - mu), axis=-1, keepdims=True)
        y = (h - mu) * lax.rsqrt(var + 1e-5) * lw_r[0] + lb_r[0]
        o_r[0] = jnp.maximum(y, 0.0)

    return pl.pallas_call(
        body,
        grid=(2,),
        in_specs=[
            pl.BlockSpec((1, NP, C), lambda t: (t, 0, 0)),
            pl.BlockSpec((1, _NS, NP // 128, 128), lambda t: (t, 0, 0, 0)),
            pl.BlockSpec((1, N, C), lambda t: (t, 0, 0)),
            pl.BlockSpec((1, C, C), lambda t: (t, 0, 0)),
            pl.BlockSpec((1, 1, C), lambda t: (t, 0, 0)),
            pl.BlockSpec((1, C, C), lambda t: (t, 0, 0)),
            pl.BlockSpec((1, 1, C), lambda t: (t, 0, 0)),
            pl.BlockSpec((1, 1, C), lambda t: (t, 0, 0)),
        ],
        out_specs=pl.BlockSpec((1, N, C), lambda t: (t, 0, 0)),
        out_shape=jax.ShapeDtypeStruct((2, N, C), jnp.float32),
    )(agg, cnt, xdst2, wlt2, bl2, wrt2, lnw2, lnb2)


def kernel(x_user, x_item, ei_ui, ei_iu,
           Wl_0_ui, bl_0_ui, Wr_0_ui, Wl_0_iu, bl_0_iu, Wr_0_iu,
           ln_w_0_user, ln_b_0_user, ln_w_0_item, ln_b_0_item,
           Wl_1_ui, bl_1_ui, Wr_1_ui, Wl_1_iu, bl_1_iu, Wr_1_iu,
           ln_w_1_user, ln_b_1_user, ln_w_1_item, ln_b_1_item):
    N, C = x_user.shape
    E = ei_ui.shape[1]
    ept = -(-E // (_NS * _K)) * _K
    S = ept // _K
    tot = ept * _NS
    NP = ((N + 1 + 127) // 128) * 128

    def prep(ei):
        pad = tot - E
        src = jnp.concatenate([ei[0], jnp.zeros((pad,), jnp.int32)])
        dst = jnp.concatenate([ei[1], jnp.full((pad,), N, jnp.int32)])
        return src.reshape(_NS, S, _K), dst.reshape(_NS, S, _K)

    su, du = prep(ei_ui)
    si, di = prep(ei_iu)
    src2 = jnp.stack([su, si])
    dst2 = jnp.stack([du, di])
    zf = jnp.zeros((NP, C), jnp.float32)
    zc1 = jnp.zeros((NP // 128, 128), jnp.float32)

    cnt = _sc_cnt_call(dst2.reshape(_NC, _NS, -1), zc1)
    agg0 = _sc_agg_call(jnp.stack([x_user, x_item]), src2, dst2, zf)
    out0 = _tc_layer(
        agg0, cnt, jnp.stack([x_item, x_user]),
        jnp.stack([Wl_0_ui.T, Wl_0_iu.T]),
        jnp.stack([bl_0_ui, bl_0_iu])[:, None, :],
        jnp.stack([Wr_0_ui.T, Wr_0_iu.T]),
        jnp.stack([ln_w_0_item, ln_w_0_user])[:, None, :],
        jnp.stack([ln_b_0_item, ln_b_0_user])[:, None, :],
    )
    x_item1, x_user1 = out0[0], out0[1]

    agg1 = _sc_agg_call(jnp.stack([x_user1, x_item1]), src2, dst2, zf)
    out1 = _tc_layer(
        agg1, cnt, jnp.stack([x_item1, x_user1]),
        jnp.stack([Wl_1_ui.T, Wl_1_iu.T]),
        jnp.stack([bl_1_ui, bl_1_iu])[:, None, :],
        jnp.stack([Wr_1_ui.T, Wr_1_iu.T]),
        jnp.stack([ln_w_1_item, ln_w_1_user])[:, None, :],
        jnp.stack([ln_b_1_item, ln_b_1_user])[:, None, :],
    )
    return (out1[1], out1[0])


# speedup vs baseline: 1.5623x; 1.0432x over previous
import jax
import jax.numpy as jnp
from jax import lax
from jax.experimental import pallas as pl
from jax.experimental.pallas import tpu as pltpu
from jax.experimental.pallas import tpu_sc as plsc

_NC = 2
_NS = 16
_K = 128


def _sc_agg_call(x2, src2, dst2, zf):
    _, _, C = x2.shape
    S = src2.shape[2]
    NP = zf.shape[0]
    R = NP // _NS
    mesh = plsc.VectorSubcoreMesh(core_axis_name="c", subcore_axis_name="s")

    outs = [jax.ShapeDtypeStruct((_NC, NP, C), jnp.float32)]
    scratch = [
        pltpu.VMEM((S, _K), jnp.int32),
        pltpu.VMEM((S, _K), jnp.int32),
        pltpu.VMEM((_K, C), jnp.float32),
        pltpu.VMEM_SHARED((NP, C), jnp.float32),
        pltpu.SemaphoreType.DMA,
    ]

    def kbody(x2_r, src_r, dst_r, zf_r, agg_o, src_v, dst_v, gbuf, acc_sh,
              sem):
        c = lax.axis_index("c")
        s = lax.axis_index("s")
        row0 = s * R

        pltpu.sync_copy(src_r.at[c, s], src_v)
        pltpu.sync_copy(dst_r.at[c, s], dst_v)
        pltpu.sync_copy(zf_r.at[pl.ds(row0, R)], acc_sh.at[pl.ds(row0, R)])
        plsc.subcore_barrier()

        xsrc = x2_r.at[c]

        def step(j, carry):
            pltpu.async_copy(xsrc.at[src_v.at[j]], gbuf, sem).wait()
            pltpu.sync_copy(gbuf, acc_sh.at[dst_v.at[j]], add=True)
            return carry

        lax.fori_loop(0, S, step, 0)
        plsc.subcore_barrier()

        pltpu.sync_copy(acc_sh.at[pl.ds(row0, R)], agg_o.at[c, pl.ds(row0, R)])

    fn = pl.kernel(kbody, out_type=tuple(outs), mesh=mesh,
                   scratch_types=tuple(scratch))
    return fn(x2, src2, dst2, zf)[0]


def _sc_cnt_call(dstf, zc1):
    ept = dstf.shape[2]
    NR = zc1.shape[0]
    mesh = plsc.VectorSubcoreMesh(core_axis_name="c", subcore_axis_name="s")

    def kbody(dst_r, zc_r, cntp_o, dst_v, cnt_v):
        c = lax.axis_index("c")
        s = lax.axis_index("s")
        pltpu.sync_copy(dst_r.at[c, s], dst_v)
        pltpu.sync_copy(zc_r, cnt_v)
        onesv = jnp.ones((16,), jnp.float32)

        def step(i, carry):
            idx = dst_v[pl.ds(i * 16, 16)]
            plsc.addupdate_scatter(cnt_v, [idx >> 7, idx & 127], onesv)
            return carry

        lax.fori_loop(0, ept // 16, step, 0)
        pltpu.sync_copy(cnt_v, cntp_o.at[c, s])

    fn = pl.kernel(
        kbody,
        out_type=(jax.ShapeDtypeStruct((_NC, _NS, NR, 128), jnp.float32),),
        mesh=mesh,
        scratch_types=(
            pltpu.VMEM((ept,), jnp.int32),
            pltpu.VMEM((NR, 128), jnp.float32),
        ),
        compiler_params=pltpu.CompilerParams(needs_layout_passes=False),
    )
    return fn(dstf, zc1)[0]


def _tc_layer(agg, cnt, xdst2, wlt2, bl2, wrt2, lnw2, lnb2):
    _, N, C = xdst2.shape
    NP = agg.shape[1]

    NR = NP // 128

    def body(agg_r, cnt_r, x_r, wl_r, bl_r, wr_r, lw_r, lb_r, o_r):
        a3 = agg_r[0].reshape(NR, 128, C)
        cv = jnp.sum(cnt_r[0], axis=0)
        inv = 1.0 / jnp.maximum(cv, 1.0)
        m = (a3 * inv[:, :, None]).reshape(NP, C)[:N]
        h = (jnp.dot(m, wl_r[0], preferred_element_type=jnp.float32)
             + bl_r[0]
             + jnp.dot(x_r[0], wr_r[0], preferred_element_type=jnp.float32))
        mu = jnp.mean(h, axis=-1, keepdims=True)
        var = jnp.mean(jnp.square(h - mu), axis=-1, keepdims=True)
        y = (h - mu) * lax.rsqrt(var + 1e-5) * lw_r[0] + lb_r[0]
        o_r[0] = jnp.maximum(y, 0.0)

    return pl.pallas_call(
        body,
        grid=(2,),
        in_specs=[
            pl.BlockSpec((1, NP, C), lambda t: (t, 0, 0)),
            pl.BlockSpec((1, _NS, NP // 128, 128), lambda t: (t, 0, 0, 0)),
            pl.BlockSpec((1, N, C), lambda t: (t, 0, 0)),
            pl.BlockSpec((1, C, C), lambda t: (t, 0, 0)),
            pl.BlockSpec((1, 1, C), lambda t: (t, 0, 0)),
            pl.BlockSpec((1, C, C), lambda t: (t, 0, 0)),
            pl.BlockSpec((1, 1, C), lambda t: (t, 0, 0)),
            pl.BlockSpec((1, 1, C), lambda t: (t, 0, 0)),
        ],
        out_specs=pl.BlockSpec((1, N, C), lambda t: (t, 0, 0)),
        out_shape=jax.ShapeDtypeStruct((2, N, C), jnp.float32),
    )(agg, cnt, xdst2, wlt2, bl2, wrt2, lnw2, lnb2)


def kernel(x_user, x_item, ei_ui, ei_iu,
           Wl_0_ui, bl_0_ui, Wr_0_ui, Wl_0_iu, bl_0_iu, Wr_0_iu,
           ln_w_0_user, ln_b_0_user, ln_w_0_item, ln_b_0_item,
           Wl_1_ui, bl_1_ui, Wr_1_ui, Wl_1_iu, bl_1_iu, Wr_1_iu,
           ln_w_1_user, ln_b_1_user, ln_w_1_item, ln_b_1_item):
    N, C = x_user.shape
    E = ei_ui.shape[1]
    ept = -(-E // (_NS * _K)) * _K
    S = ept // _K
    tot = ept * _NS
    NP = ((N + 1 + 127) // 128) * 128

    pad = tot - E
    pad_dst = N + (jnp.arange(pad, dtype=jnp.int32) % (NP - N))

    def prep(ei):
        src = jnp.concatenate([ei[0], jnp.zeros((pad,), jnp.int32)])
        dst = jnp.concatenate([ei[1], pad_dst])
        return src.reshape(_NS, S, _K), dst.reshape(_NS, S, _K)

    su, du = prep(ei_ui)
    si, di = prep(ei_iu)
    src2 = jnp.stack([su, si])
    dst2 = jnp.stack([du, di])
    zf = jnp.zeros((NP, C), jnp.float32)
    zc1 = jnp.zeros((NP // 128, 128), jnp.float32)

    cnt = _sc_cnt_call(dst2.reshape(_NC, _NS, -1), zc1)
    agg0 = _sc_agg_call(jnp.stack([x_user, x_item]), src2, dst2, zf)
    out0 = _tc_layer(
        agg0, cnt, jnp.stack([x_item, x_user]),
        jnp.stack([Wl_0_ui.T, Wl_0_iu.T]),
        jnp.stack([bl_0_ui, bl_0_iu])[:, None, :],
        jnp.stack([Wr_0_ui.T, Wr_0_iu.T]),
        jnp.stack([ln_w_0_item, ln_w_0_user])[:, None, :],
        jnp.stack([ln_b_0_item, ln_b_0_user])[:, None, :],
    )
    x_item1, x_user1 = out0[0], out0[1]

    agg1 = _sc_agg_call(jnp.stack([x_user1, x_item1]), src2, dst2, zf)
    out1 = _tc_layer(
        agg1, cnt, jnp.stack([x_item1, x_user1]),
        jnp.stack([Wl_1_ui.T, Wl_1_iu.T]),
        jnp.stack([bl_1_ui, bl_1_iu])[:, None, :],
        jnp.stack([Wr_1_ui.T, Wr_1_iu.T]),
        jnp.stack([ln_w_1_item, ln_w_1_user])[:, None, :],
        jnp.stack([ln_b_1_item, ln_b_1_user])[:, None, :],
    )
    return (out1[1], out1[0])
